# baseline (speedup 1.0000x reference)
import functools

import jax
import jax.numpy as jnp
from jax.experimental import pallas as pl
from jax.experimental.pallas import tpu as pltpu

L = 2048
C = 1024
H = 16
K = 64
HALF_S = 16
S = 2 * HALF_S + 1
HS = H * S // 2
G = 8
MAX_FREQ = 16.0
MIN_FREQ = 1.0

LBLK_A = 256
LBLK_B = 64
LBLK_C = 256


def _silu(v):
    return v * jax.nn.sigmoid(v)


def _proj_kernel(x_ref, ww_ref, wb_ref, kw_ref, kb_ref,
                 pos_ref, we_ref, wo_ref):
    i = pl.program_id(0)
    xb = x_ref[...]
    wave = _silu(
        jax.lax.dot_general(xb, ww_ref[...], (((1,), (1,)), ((), ())),
                            precision=jax.lax.Precision.DEFAULT)
        + wb_ref[...])
    freq = jax.nn.sigmoid(wave[:, :H]) * (MAX_FREQ - MIN_FREQ) + MIN_FREQ
    phase = jnp.tanh(wave[:, H:]) * MAX_FREQ
    freq_avg = jnp.mean(freq, axis=1, keepdims=True)
    phase_avg = jnp.mean(phase, axis=1, keepdims=True)
    s_off = (jax.lax.broadcasted_iota(jnp.int32, (1, S), 1)
             .astype(jnp.float32) - HALF_S)
    base = ((i * LBLK_A).astype(jnp.float32)
            + jax.lax.broadcasted_iota(jnp.int32, (LBLK_A, 1), 0)
            .astype(jnp.float32))
    offsets = phase_avg + s_off * freq_avg
    posf = jnp.clip(jnp.round(base + offsets), 0, L - 1)
    pos_ref[...] = posf.astype(jnp.int32)
    kb = _silu(
        jax.lax.dot_general(xb.astype(jnp.bfloat16), kw_ref[...],
                            (((1,), (1,)), ((), ())),
                            preferred_element_type=jnp.float32)
        + kb_ref[...])
    we_ref[...] = kb[:, :HS]
    wo_ref[...] = kb[:, HS:]


def _gather_kernel(x_hbm_ref, pos_ref, we_ref, wo_ref, out_ref,
                   x_ref, copy_sem):
    @pl.when(pl.program_id(0) == 0)
    def _():
        pltpu.make_async_copy(x_hbm_ref, x_ref, copy_sem).start()
        pltpu.make_async_copy(x_hbm_ref, x_ref, copy_sem).wait()

    lane = jax.lax.broadcasted_iota(jnp.int32, (G, 128), 1)
    even_lane = lane < K

    def body(t, _):
        we = we_ref[t]
        wo = wo_ref[t]
        accs = [jnp.zeros((G, 128), dtype=jnp.float32) for _ in range(2)]
        for s in range(S):
            p = pos_ref[t, s]
            row = x_ref[p]
            wexp = jnp.where(even_lane, we[:, s:s + 1], wo[:, s:s + 1])
            accs[s % 2] = accs[s % 2] + wexp * row
        out_ref[t] = accs[0] + accs[1]
        return 0

    jax.lax.fori_loop(0, LBLK_B, body, 0)


def _out_kernel(h_ref, ow_ref, o_ref):
    o_ref[...] = _silu(
        jax.lax.dot_general(h_ref[...].astype(jnp.bfloat16), ow_ref[...],
                            (((1,), (1,)), ((), ())),
                            preferred_element_type=jnp.float32))


@functools.partial(jax.jit, static_argnames=("interpret",))
def kernel(x, wave_w, wave_b, kernel_w, kernel_b, out_w, interpret=False):
    b, l, c = x.shape
    x2 = x.reshape(l, c)
    kw4 = kernel_w.reshape(H, K, c)[:, :S]
    kw_r = jnp.concatenate([kw4[0::2], kw4[1::2]], axis=0).reshape(2 * HS, c)
    kb4 = kernel_b.reshape(H, K)[:, :S]
    kb_r = jnp.concatenate([kb4[0::2], kb4[1::2]], axis=0).reshape(2 * HS)

    pos, we, wo = pl.pallas_call(
        _proj_kernel,
        grid=(l // LBLK_A,),
        in_specs=[
            pl.BlockSpec((LBLK_A, c), lambda i: (i, 0)),
            pl.BlockSpec((2 * H, c), lambda i: (0, 0)),
            pl.BlockSpec((1, 2 * H), lambda i: (0, 0)),
            pl.BlockSpec((2 * HS, c), lambda i: (0, 0)),
            pl.BlockSpec((1, 2 * HS), lambda i: (0, 0)),
        ],
        out_specs=[
            pl.BlockSpec((LBLK_A, S), lambda i: (i, 0)),
            pl.BlockSpec((LBLK_A, HS), lambda i: (i, 0)),
            pl.BlockSpec((LBLK_A, HS), lambda i: (i, 0)),
        ],
        out_shape=[
            jax.ShapeDtypeStruct((l, S), jnp.int32),
            jax.ShapeDtypeStruct((l, HS), jnp.float32),
            jax.ShapeDtypeStruct((l, HS), jnp.float32),
        ],
        interpret=interpret,
    )(x2, wave_w, wave_b.reshape(1, 2 * H), kw_r.astype(jnp.bfloat16),
      kb_r.reshape(1, 2 * HS))

    x4 = x2.reshape(l, G, 128)
    we3 = we.reshape(l, G, S)
    wo3 = wo.reshape(l, G, S)

    out_h = pl.pallas_call(
        _gather_kernel,
        grid=(l // LBLK_B,),
        in_specs=[
            pl.BlockSpec(memory_space=pl.ANY),
            pl.BlockSpec((LBLK_B, S), lambda i: (i, 0),
                         memory_space=pltpu.SMEM),
            pl.BlockSpec((LBLK_B, G, S), lambda i: (i, 0, 0)),
            pl.BlockSpec((LBLK_B, G, S), lambda i: (i, 0, 0)),
        ],
        out_specs=pl.BlockSpec((LBLK_B, G, 128), lambda i: (i, 0, 0)),
        out_shape=jax.ShapeDtypeStruct((l, G, 128), jnp.float32),
        scratch_shapes=[
            pltpu.VMEM((l, G, 128), jnp.float32),
            pltpu.SemaphoreType.DMA,
        ],
        interpret=interpret,
    )(x4, pos, we3, wo3)

    out = pl.pallas_call(
        _out_kernel,
        grid=(l // LBLK_C,),
        in_specs=[
            pl.BlockSpec((LBLK_C, c), lambda i: (i, 0)),
            pl.BlockSpec((c, c), lambda i: (0, 0)),
        ],
        out_specs=pl.BlockSpec((LBLK_C, c), lambda i: (i, 0)),
        out_shape=jax.ShapeDtypeStruct((l, c), jnp.float32),
        interpret=interpret,
    )(out_h.reshape(l, c), out_w.astype(jnp.bfloat16))

    return out.reshape(b, l, c)


# speedup vs baseline: 1.0199x; 1.0199x over previous
import functools

import jax
import jax.numpy as jnp
from jax.experimental import pallas as pl
from jax.experimental.pallas import tpu as pltpu

L = 2048
C = 1024
H = 16
K = 64
HALF_S = 16
S = 2 * HALF_S + 1
HS = H * S // 2
G = 8
MAX_FREQ = 16.0
MIN_FREQ = 1.0

LBLK_A = 512
LBLK_B = 128


def _silu(v):
    return v * jax.nn.sigmoid(v)


def _proj_kernel(x_ref, ww_ref, wb_ref, kw_ref, kb_ref,
                 pos_ref, we_ref, wo_ref):
    i = pl.program_id(0)
    xb = x_ref[...]
    wave = _silu(
        jax.lax.dot_general(xb, ww_ref[...], (((1,), (1,)), ((), ())),
                            precision=jax.lax.Precision.DEFAULT)
        + wb_ref[...])
    freq = jax.nn.sigmoid(wave[:, :H]) * (MAX_FREQ - MIN_FREQ) + MIN_FREQ
    phase = jnp.tanh(wave[:, H:]) * MAX_FREQ
    freq_avg = jnp.mean(freq, axis=1, keepdims=True)
    phase_avg = jnp.mean(phase, axis=1, keepdims=True)
    s_off = (jax.lax.broadcasted_iota(jnp.int32, (1, S), 1)
             .astype(jnp.float32) - HALF_S)
    base = ((i * LBLK_A).astype(jnp.float32)
            + jax.lax.broadcasted_iota(jnp.int32, (LBLK_A, 1), 0)
            .astype(jnp.float32))
    offsets = phase_avg + s_off * freq_avg
    posf = jnp.clip(jnp.round(base + offsets), 0, L - 1)
    pos_ref[...] = posf.astype(jnp.int32)
    kb = _silu(
        jax.lax.dot_general(xb.astype(jnp.bfloat16), kw_ref[...],
                            (((1,), (1,)), ((), ())),
                            preferred_element_type=jnp.float32)
        + kb_ref[...])
    we_ref[...] = kb[:, :HS].reshape(LBLK_A, G, S)
    wo_ref[...] = kb[:, HS:].reshape(LBLK_A, G, S)


def _gather_out_kernel(x_hbm_ref, pos_ref, we_ref, wo_ref, ow_ref,
                       o_ref, x_ref, hb_ref, copy_sem):
    @pl.when(pl.program_id(0) == 0)
    def _():
        pltpu.make_async_copy(x_hbm_ref, x_ref, copy_sem).start()
        pltpu.make_async_copy(x_hbm_ref, x_ref, copy_sem).wait()

    lane = jax.lax.broadcasted_iota(jnp.int32, (G, 128), 1)
    even_lane = lane < K

    def body(t, _):
        we = we_ref[t]
        wo = wo_ref[t]
        accs = [jnp.zeros((G, 128), dtype=jnp.float32) for _ in range(2)]
        for s in range(S):
            p = pos_ref[t, s]
            row = x_ref[p]
            wexp = jnp.where(even_lane, we[:, s:s + 1], wo[:, s:s + 1])
            accs[s % 2] = accs[s % 2] + wexp * row
        hb_ref[t] = accs[0] + accs[1]
        return 0

    jax.lax.fori_loop(0, LBLK_B, body, 0)

    hb = hb_ref[...].astype(jnp.bfloat16)
    acc = None
    for g in range(G):
        hg = hb[:, g, :]
        og = ow_ref[:, g * 128:(g + 1) * 128]
        d = jax.lax.dot_general(hg, og, (((1,), (1,)), ((), ())),
                                preferred_element_type=jnp.float32)
        acc = d if acc is None else acc + d
    o_ref[...] = _silu(acc)


@functools.partial(jax.jit, static_argnames=("interpret",))
def kernel(x, wave_w, wave_b, kernel_w, kernel_b, out_w, interpret=False):
    b, l, c = x.shape
    x2 = x.reshape(l, c)
    kw4 = kernel_w.reshape(H, K, c)[:, :S]
    kw_r = jnp.concatenate([kw4[0::2], kw4[1::2]], axis=0).reshape(2 * HS, c)
    kb4 = kernel_b.reshape(H, K)[:, :S]
    kb_r = jnp.concatenate([kb4[0::2], kb4[1::2]], axis=0).reshape(2 * HS)

    pos, we4, wo4 = pl.pallas_call(
        _proj_kernel,
        grid=(l // LBLK_A,),
        in_specs=[
            pl.BlockSpec((LBLK_A, c), lambda i: (i, 0)),
            pl.BlockSpec((2 * H, c), lambda i: (0, 0)),
            pl.BlockSpec((1, 2 * H), lambda i: (0, 0)),
            pl.BlockSpec((2 * HS, c), lambda i: (0, 0)),
            pl.BlockSpec((1, 2 * HS), lambda i: (0, 0)),
        ],
        out_specs=[
            pl.BlockSpec((LBLK_A, S), lambda i: (i, 0)),
            pl.BlockSpec((LBLK_A, G, S), lambda i: (i, 0, 0)),
            pl.BlockSpec((LBLK_A, G, S), lambda i: (i, 0, 0)),
        ],
        out_shape=[
            jax.ShapeDtypeStruct((l, S), jnp.int32),
            jax.ShapeDtypeStruct((l, G, S), jnp.float32),
            jax.ShapeDtypeStruct((l, G, S), jnp.float32),
        ],
        interpret=interpret,
    )(x2, wave_w, wave_b.reshape(1, 2 * H), kw_r.astype(jnp.bfloat16),
      kb_r.reshape(1, 2 * HS))

    x4 = x2.reshape(l, G, 128)

    out = pl.pallas_call(
        _gather_out_kernel,
        grid=(l // LBLK_B,),
        in_specs=[
            pl.BlockSpec(memory_space=pl.ANY),
            pl.BlockSpec((LBLK_B, S), lambda i: (i, 0),
                         memory_space=pltpu.SMEM),
            pl.BlockSpec((LBLK_B, G, S), lambda i: (i, 0, 0)),
            pl.BlockSpec((LBLK_B, G, S), lambda i: (i, 0, 0)),
            pl.BlockSpec((c, c), lambda i: (0, 0)),
        ],
        out_specs=pl.BlockSpec((LBLK_B, c), lambda i: (i, 0)),
        out_shape=jax.ShapeDtypeStruct((l, c), jnp.float32),
        scratch_shapes=[
            pltpu.VMEM((l, G, 128), jnp.float32),
            pltpu.VMEM((LBLK_B, G, 128), jnp.float32),
            pltpu.SemaphoreType.DMA,
        ],
        interpret=interpret,
    )(x4, pos, we4, wo4, out_w.astype(jnp.bfloat16))

    return out.reshape(b, l, c)


# speedup vs baseline: 1.2051x; 1.1815x over previous
import functools

import jax
import jax.numpy as jnp
from jax.experimental import pallas as pl
from jax.experimental.pallas import tpu as pltpu

L = 2048
C = 1024
H = 16
K = 64
HALF_S = 16
S = 2 * HALF_S + 1
HS = H * S // 2
G = 8
MAX_FREQ = 16.0
MIN_FREQ = 1.0

LBLK_A = 512
LBLK_B = 128


def _silu(v):
    return v * jax.nn.sigmoid(v)


def _proj_kernel(x_ref, ww_ref, wb_ref, kw_ref, kb_ref,
                 pos_ref, we_ref, wo_ref):
    i = pl.program_id(0)
    xb = x_ref[...]
    wave = _silu(
        jax.lax.dot_general(xb, ww_ref[...], (((1,), (1,)), ((), ())),
                            precision=jax.lax.Precision.DEFAULT)
        + wb_ref[...])
    freq = jax.nn.sigmoid(wave[:, :H]) * (MAX_FREQ - MIN_FREQ) + MIN_FREQ
    phase = jnp.tanh(wave[:, H:]) * MAX_FREQ
    freq_avg = jnp.mean(freq, axis=1, keepdims=True)
    phase_avg = jnp.mean(phase, axis=1, keepdims=True)
    s_off = (jax.lax.broadcasted_iota(jnp.int32, (1, S), 1)
             .astype(jnp.float32) - HALF_S)
    base = ((i * LBLK_A).astype(jnp.float32)
            + jax.lax.broadcasted_iota(jnp.int32, (LBLK_A, 1), 0)
            .astype(jnp.float32))
    offsets = phase_avg + s_off * freq_avg
    posf = jnp.clip(jnp.round(base + offsets), 0, L - 1)
    pos_ref[...] = posf.astype(jnp.int32)
    kb = _silu(
        jax.lax.dot_general(xb.astype(jnp.bfloat16), kw_ref[...],
                            (((1,), (1,)), ((), ())),
                            preferred_element_type=jnp.float32)
        + kb_ref[...])
    we_ref[...] = kb[:, :HS].reshape(LBLK_A, G, S)
    wo_ref[...] = kb[:, HS:].reshape(LBLK_A, G, S)


def _gather_out_kernel(x_hbm_ref, pos_ref, we_ref, wo_ref, ow_ref,
                       o_ref, x_ref, hb_ref, copy_sem):
    @pl.when(pl.program_id(0) == 0)
    def _():
        pltpu.make_async_copy(x_hbm_ref, x_ref, copy_sem).start()
        pltpu.make_async_copy(x_hbm_ref, x_ref, copy_sem).wait()

    lane = jax.lax.broadcasted_iota(jnp.int32, (G, 128), 1)
    even_lane = lane < K

    def body(t, _):
        we = we_ref[t]
        wo = wo_ref[t]
        accs = [jnp.zeros((G, 128), dtype=jnp.float32) for _ in range(2)]
        for s in range(S):
            p = pos_ref[t, s]
            row = x_ref[p]
            wexp = jnp.where(even_lane, we[:, s:s + 1], wo[:, s:s + 1])
            accs[s % 2] = accs[s % 2] + wexp * row
        hb_ref[t] = accs[0] + accs[1]
        return 0

    jax.lax.fori_loop(0, LBLK_B, body, 0, unroll=2)

    hb = hb_ref[...].astype(jnp.bfloat16)
    acc = None
    for g in range(G):
        hg = hb[:, g, :]
        og = ow_ref[:, g * 128:(g + 1) * 128]
        d = jax.lax.dot_general(hg, og, (((1,), (1,)), ((), ())),
                                preferred_element_type=jnp.float32)
        acc = d if acc is None else acc + d
    o_ref[...] = _silu(acc)


@functools.partial(jax.jit, static_argnames=("interpret",))
def kernel(x, wave_w, wave_b, kernel_w, kernel_b, out_w, interpret=False):
    b, l, c = x.shape
    x2 = x.reshape(l, c)
    kw4 = kernel_w.reshape(H, K, c)[:, :S]
    kw_r = jnp.concatenate([kw4[0::2], kw4[1::2]], axis=0).reshape(2 * HS, c)
    kb4 = kernel_b.reshape(H, K)[:, :S]
    kb_r = jnp.concatenate([kb4[0::2], kb4[1::2]], axis=0).reshape(2 * HS)

    pos, we4, wo4 = pl.pallas_call(
        _proj_kernel,
        grid=(l // LBLK_A,),
        in_specs=[
            pl.BlockSpec((LBLK_A, c), lambda i: (i, 0)),
            pl.BlockSpec((2 * H, c), lambda i: (0, 0)),
            pl.BlockSpec((1, 2 * H), lambda i: (0, 0)),
            pl.BlockSpec((2 * HS, c), lambda i: (0, 0)),
            pl.BlockSpec((1, 2 * HS), lambda i: (0, 0)),
        ],
        out_specs=[
            pl.BlockSpec((LBLK_A, S), lambda i: (i, 0)),
            pl.BlockSpec((LBLK_A, G, S), lambda i: (i, 0, 0)),
            pl.BlockSpec((LBLK_A, G, S), lambda i: (i, 0, 0)),
        ],
        out_shape=[
            jax.ShapeDtypeStruct((l, S), jnp.int32),
            jax.ShapeDtypeStruct((l, G, S), jnp.float32),
            jax.ShapeDtypeStruct((l, G, S), jnp.float32),
        ],
        interpret=interpret,
    )(x2, wave_w, wave_b.reshape(1, 2 * H), kw_r.astype(jnp.bfloat16),
      kb_r.reshape(1, 2 * HS))

    x4 = x2.reshape(l, G, 128)

    out = pl.pallas_call(
        _gather_out_kernel,
        grid=(l // LBLK_B,),
        in_specs=[
            pl.BlockSpec(memory_space=pl.ANY),
            pl.BlockSpec((LBLK_B, S), lambda i: (i, 0),
                         memory_space=pltpu.SMEM),
            pl.BlockSpec((LBLK_B, G, S), lambda i: (i, 0, 0)),
            pl.BlockSpec((LBLK_B, G, S), lambda i: (i, 0, 0)),
            pl.BlockSpec((c, c), lambda i: (0, 0)),
        ],
        out_specs=pl.BlockSpec((LBLK_B, c), lambda i: (i, 0)),
        out_shape=jax.ShapeDtypeStruct((l, c), jnp.float32),
        scratch_shapes=[
            pltpu.VMEM((l, G, 128), jnp.float32),
            pltpu.VMEM((LBLK_B, G, 128), jnp.float32),
            pltpu.SemaphoreType.DMA,
        ],
        interpret=interpret,
    )(x4, pos, we4, wo4, out_w.astype(jnp.bfloat16))

    return out.reshape(b, l, c)


# speedup vs baseline: 1.3291x; 1.1029x over previous
import functools

import jax
import jax.numpy as jnp
from jax.experimental import pallas as pl
from jax.experimental.pallas import tpu as pltpu

L = 2048
C = 1024
H = 16
K = 64
HALF_S = 16
S = 2 * HALF_S + 1
HS = H * S // 2
G = 8
MAX_FREQ = 16.0
MIN_FREQ = 1.0

LBLK_A = 512
LBLK_B = 128


def _silu(v):
    return v * jax.nn.sigmoid(v)


def _proj_kernel(x_ref, ww_ref, wb_ref, kw_ref, kb_ref,
                 pos_ref, we_ref, wo_ref):
    i = pl.program_id(0)
    xb = x_ref[...]
    wave = _silu(
        jax.lax.dot_general(xb, ww_ref[...], (((1,), (1,)), ((), ())),
                            precision=jax.lax.Precision.DEFAULT)
        + wb_ref[...])
    freq = jax.nn.sigmoid(wave[:, :H]) * (MAX_FREQ - MIN_FREQ) + MIN_FREQ
    phase = jnp.tanh(wave[:, H:]) * MAX_FREQ
    freq_avg = jnp.mean(freq, axis=1, keepdims=True)
    phase_avg = jnp.mean(phase, axis=1, keepdims=True)
    s_off = (jax.lax.broadcasted_iota(jnp.int32, (1, S), 1)
             .astype(jnp.float32) - HALF_S)
    base = ((i * LBLK_A).astype(jnp.float32)
            + jax.lax.broadcasted_iota(jnp.int32, (LBLK_A, 1), 0)
            .astype(jnp.float32))
    offsets = phase_avg + s_off * freq_avg
    posf = jnp.clip(jnp.round(base + offsets), 0, L - 1)
    pos_ref[...] = posf.astype(jnp.int32)
    kb = _silu(
        jax.lax.dot_general(xb.astype(jnp.bfloat16), kw_ref[...],
                            (((1,), (1,)), ((), ())),
                            preferred_element_type=jnp.float32)
        + kb_ref[...])
    we_ref[...] = kb[:, :HS].reshape(LBLK_A, G, S)
    wo_ref[...] = kb[:, HS:].reshape(LBLK_A, G, S)


def _gather_out_kernel(x_hbm_ref, pos_ref, we_ref, wo_ref, ow_ref,
                       o_ref, x_ref, hb_ref, copy_sem):
    @pl.when(pl.program_id(0) == 0)
    def _():
        pltpu.make_async_copy(x_hbm_ref, x_ref, copy_sem).start()
        pltpu.make_async_copy(x_hbm_ref, x_ref, copy_sem).wait()

    lane = jax.lax.broadcasted_iota(jnp.int32, (G, 128), 1)
    even_lane = lane < K

    def body(t, _):
        we = we_ref[t]
        wo = wo_ref[t]
        accs = [jnp.zeros((G, 128), dtype=jnp.float32) for _ in range(2)]
        for s in range(S):
            p = pos_ref[t, s]
            row = x_ref[p]
            wexp = jnp.where(even_lane, we[:, s:s + 1], wo[:, s:s + 1])
            accs[s % 2] = accs[s % 2] + wexp * row
        hb_ref[t] = accs[0] + accs[1]
        return 0

    jax.lax.fori_loop(0, LBLK_B, body, 0, unroll=4)

    hb = hb_ref[...].astype(jnp.bfloat16)
    acc = None
    for g in range(G):
        hg = hb[:, g, :]
        og = ow_ref[:, g * 128:(g + 1) * 128]
        d = jax.lax.dot_general(hg, og, (((1,), (1,)), ((), ())),
                                preferred_element_type=jnp.float32)
        acc = d if acc is None else acc + d
    o_ref[...] = _silu(acc)


@functools.partial(jax.jit, static_argnames=("interpret",))
def kernel(x, wave_w, wave_b, kernel_w, kernel_b, out_w, interpret=False):
    b, l, c = x.shape
    x2 = x.reshape(l, c)
    kw4 = kernel_w.reshape(H, K, c)[:, :S]
    kw_r = jnp.concatenate([kw4[0::2], kw4[1::2]], axis=0).reshape(2 * HS, c)
    kb4 = kernel_b.reshape(H, K)[:, :S]
    kb_r = jnp.concatenate([kb4[0::2], kb4[1::2]], axis=0).reshape(2 * HS)

    pos, we4, wo4 = pl.pallas_call(
        _proj_kernel,
        grid=(l // LBLK_A,),
        in_specs=[
            pl.BlockSpec((LBLK_A, c), lambda i: (i, 0)),
            pl.BlockSpec((2 * H, c), lambda i: (0, 0)),
            pl.BlockSpec((1, 2 * H), lambda i: (0, 0)),
            pl.BlockSpec((2 * HS, c), lambda i: (0, 0)),
            pl.BlockSpec((1, 2 * HS), lambda i: (0, 0)),
        ],
        out_specs=[
            pl.BlockSpec((LBLK_A, S), lambda i: (i, 0)),
            pl.BlockSpec((LBLK_A, G, S), lambda i: (i, 0, 0)),
            pl.BlockSpec((LBLK_A, G, S), lambda i: (i, 0, 0)),
        ],
        out_shape=[
            jax.ShapeDtypeStruct((l, S), jnp.int32),
            jax.ShapeDtypeStruct((l, G, S), jnp.float32),
            jax.ShapeDtypeStruct((l, G, S), jnp.float32),
        ],
        interpret=interpret,
    )(x2, wave_w, wave_b.reshape(1, 2 * H), kw_r.astype(jnp.bfloat16),
      kb_r.reshape(1, 2 * HS))

    x4 = x2.reshape(l, G, 128)

    out = pl.pallas_call(
        _gather_out_kernel,
        grid=(l // LBLK_B,),
        in_specs=[
            pl.BlockSpec(memory_space=pl.ANY),
            pl.BlockSpec((LBLK_B, S), lambda i: (i, 0),
                         memory_space=pltpu.SMEM),
            pl.BlockSpec((LBLK_B, G, S), lambda i: (i, 0, 0)),
            pl.BlockSpec((LBLK_B, G, S), lambda i: (i, 0, 0)),
            pl.BlockSpec((c, c), lambda i: (0, 0)),
        ],
        out_specs=pl.BlockSpec((LBLK_B, c), lambda i: (i, 0)),
        out_shape=jax.ShapeDtypeStruct((l, c), jnp.float32),
        scratch_shapes=[
            pltpu.VMEM((l, G, 128), jnp.float32),
            pltpu.VMEM((LBLK_B, G, 128), jnp.float32),
            pltpu.SemaphoreType.DMA,
        ],
        interpret=interpret,
    )(x4, pos, we4, wo4, out_w.astype(jnp.bfloat16))

    return out.reshape(b, l, c)
